# baseline (speedup 1.0000x reference)
import functools

import jax
import jax.numpy as jnp
from jax import lax
from jax.experimental import pallas as pl
from jax.experimental.pallas import tpu as pltpu
from jax.experimental.pallas import tpu_sc as plsc

N_NODES = 10000
DIM = 128
N_EDGES = 320000

L = 16
NC = 2
NS = 16
NW = NC * NS
DPT = DIM // NW
PPT = DPT // 2
CHUNK = 8000
NCHUNK = N_EDGES // CHUNK
NBLK = CHUNK // L

_mesh = plsc.VectorSubcoreMesh(core_axis_name="c", subcore_axis_name="s")


@functools.partial(
    pl.kernel,
    mesh=_mesh,
    out_type=jax.ShapeDtypeStruct((DIM * N_NODES,), jnp.float32),
    compiler_params=pltpu.CompilerParams(needs_layout_passes=False),
    scratch_types=(
        [pltpu.VMEM((N_NODES,), jnp.int32) for _ in range(PPT)]
        + [pltpu.VMEM((N_NODES,), jnp.float32) for _ in range(DPT)]
        + [
            pltpu.VMEM((CHUNK,), jnp.int32),
            pltpu.VMEM((CHUNK,), jnp.float32),
            pltpu.VMEM((CHUNK,), jnp.int32),
            pltpu.VMEM((CHUNK,), jnp.float32),
            pltpu.SemaphoreType.DMA,
            pltpu.SemaphoreType.DMA,
            pltpu.SemaphoreType.DMA,
        ]
    ),
)
def _sc_aggregate(rc, vals, hp, out,
                  t0, t1, a0, a1, a2, a3, rc0, val0, rc1, val1,
                  sem0, sem1, tsem):
    wid = lax.axis_index("s") * NC + lax.axis_index("c")
    pbase = wid * PPT * N_NODES
    base = wid * DPT * N_NODES
    tables = (t0, t1)
    accs = (a0, a1, a2, a3)

    bufs = ((rc0, val0, sem0), (rc1, val1, sem1))

    def _start(c, bi):
        cb, vb, sem = bufs[bi]
        off = c * CHUNK
        pltpu.async_copy(rc.at[pl.ds(off, CHUNK)], cb, sem)
        pltpu.async_copy(vals.at[pl.ds(off, CHUNK)], vb, sem)

    def _wait(bi):
        cb, vb, sem = bufs[bi]
        pltpu.make_async_copy(rc.at[pl.ds(0, CHUNK)], cb, sem).wait()
        pltpu.make_async_copy(vals.at[pl.ds(0, CHUNK)], vb, sem).wait()

    tcopies = [
        pltpu.async_copy(hp.at[pl.ds(pbase + p * N_NODES, N_NODES)], tables[p], tsem)
        for p in range(PPT)
    ]
    _start(0, 0)

    zeros = jnp.zeros((L,), jnp.float32)

    @plsc.parallel_loop(0, N_NODES // L, 1, unroll=8)
    def _zero(i):
        for d in range(DPT):
            accs[d][pl.ds(i * L, L)] = zeros

    for c in tcopies:
        c.wait()

    lowmask = jnp.full((L,), 0x3FFF, jnp.int32)
    himask = jnp.full((L,), -65536, jnp.int32)

    def _process(bi):
        cb, vb, _ = bufs[bi]

        @plsc.parallel_loop(0, NBLK, 1, unroll=4)
        def _blk(b):
            s = b * L
            cr = cb[pl.ds(s, L)]
            vv = vb[pl.ds(s, L)]
            ci = cr & lowmask
            ri = lax.shift_right_logical(cr, 14)
            for p in range(PPT):
                g = plsc.load_gather(tables[p], [ci])
                glo = plsc.bitcast(lax.shift_left(g, 16), jnp.float32)
                ghi = plsc.bitcast(g & himask, jnp.float32)
                plsc.addupdate_scatter(accs[2 * p], [ri], glo * vv)
                plsc.addupdate_scatter(accs[2 * p + 1], [ri], ghi * vv)

    def _outer(cc, carry):
        c0 = cc * 2
        _start(c0 + 1, 1)
        _wait(0)
        _process(0)

        @pl.when(c0 + 2 < NCHUNK)
        def _():
            _start(c0 + 2, 0)

        _wait(1)
        _process(1)
        return carry

    lax.fori_loop(0, NCHUNK // 2, _outer, 0)

    for d in range(DPT):
        pltpu.sync_copy(accs[d], out.at[pl.ds(base + d * N_NODES, N_NODES)])


def _pack_bf16_pair(ulo, uhi):
    plo = lax.shift_right_logical(ulo + jnp.uint32(0x8000), jnp.uint32(16))
    phi = (uhi + jnp.uint32(0x8000)) & jnp.uint32(0xFFFF0000)
    return lax.bitcast_convert_type(phi | plo, jnp.int32)


def _tc_linear_relu_packT(aggT, Wlo, Whi, blo, bhi):
    def body(agg_ref, wlo_ref, whi_ref, blo_ref, bhi_ref, out_ref):
        ylo = jnp.maximum(
            lax.dot_general(wlo_ref[...], agg_ref[...], (((1,), (0,)), ((), ())),
                            preferred_element_type=jnp.float32) + blo_ref[...], 0.0)
        yhi = jnp.maximum(
            lax.dot_general(whi_ref[...], agg_ref[...], (((1,), (0,)), ((), ())),
                            preferred_element_type=jnp.float32) + bhi_ref[...], 0.0)
        out_ref[...] = _pack_bf16_pair(
            lax.bitcast_convert_type(ylo, jnp.uint32),
            lax.bitcast_convert_type(yhi, jnp.uint32))

    return pl.pallas_call(
        body,
        out_shape=jax.ShapeDtypeStruct((DIM // 2, N_NODES), jnp.int32),
    )(aggT, Wlo, Whi, blo, bhi)


def _tc_linear_relu_final(aggT, W, b_row):
    def body(agg_ref, w_ref, b_ref, out_ref):
        acc = lax.dot_general(
            agg_ref[...], w_ref[...], (((0,), (1,)), ((), ())),
            preferred_element_type=jnp.float32)
        out_ref[...] = jnp.maximum(acc + b_ref[...], 0.0)

    return pl.pallas_call(
        body,
        out_shape=jax.ShapeDtypeStruct((N_NODES, DIM), jnp.float32),
    )(aggT, W, b_row)


def kernel(edge_index, adj_vals, emb, W1, b1, W2, b2):
    row = edge_index[0].astype(jnp.int32)
    col = edge_index[1].astype(jnp.int32)
    rc = (row << 14) | col

    u = lax.bitcast_convert_type(emb, jnp.uint32)
    hp1 = _pack_bf16_pair(u[:, 0::2], u[:, 1::2]).T

    agg1 = _sc_aggregate(rc, adj_vals, hp1.reshape(-1)).reshape(DIM, N_NODES)
    hp2 = _tc_linear_relu_packT(agg1, W1[0::2], W1[1::2],
                                b1[0::2].reshape(DIM // 2, 1),
                                b1[1::2].reshape(DIM // 2, 1))
    agg2 = _sc_aggregate(rc, adj_vals, hp2.reshape(-1)).reshape(DIM, N_NODES)
    out = _tc_linear_relu_final(agg2, W2, b2.reshape(1, DIM))
    return out


# speedup vs baseline: 1.4670x; 1.4670x over previous
import functools

import jax
import jax.numpy as jnp
from jax import lax
from jax.experimental import pallas as pl
from jax.experimental.pallas import tpu as pltpu
from jax.experimental.pallas import tpu_sc as plsc

N_NODES = 10000
DIM = 128
N_EDGES = 320000

L = 16
NC = 2
NS = 16
NW = NC * NS
DPT = DIM // NW
PPT = DPT // 2
CHUNK = 8000
NCHUNK = N_EDGES // CHUNK
NBLK = CHUNK // L

_mesh = plsc.VectorSubcoreMesh(core_axis_name="c", subcore_axis_name="s")


@functools.partial(
    pl.kernel,
    mesh=_mesh,
    out_type=jax.ShapeDtypeStruct((DIM * N_NODES,), jnp.float32),
    compiler_params=pltpu.CompilerParams(needs_layout_passes=False),
    scratch_types=(
        [pltpu.VMEM((N_NODES,), jnp.int32) for _ in range(PPT)]
        + [pltpu.VMEM((N_NODES,), jnp.float32) for _ in range(DPT)]
        + [
            pltpu.VMEM((CHUNK,), jnp.int32),
            pltpu.VMEM((CHUNK,), jnp.float32),
            pltpu.VMEM((CHUNK,), jnp.int32),
            pltpu.VMEM((CHUNK,), jnp.float32),
            pltpu.SemaphoreType.DMA,
            pltpu.SemaphoreType.DMA,
            pltpu.SemaphoreType.DMA,
        ]
    ),
)
def _sc_aggregate(rc, vals, hp, out,
                  t0, t1, a0, a1, a2, a3, rc0, val0, rc1, val1,
                  sem0, sem1, tsem):
    wid = lax.axis_index("s") * NC + lax.axis_index("c")
    pbase = wid * PPT * N_NODES
    tables = (t0, t1)
    accs = (a0, a1, a2, a3)
    acc_dims = [PPT * wid + p + (DIM // 2) * h for p in range(PPT) for h in range(2)]

    bufs = ((rc0, val0, sem0), (rc1, val1, sem1))

    def _start(c, bi):
        cb, vb, sem = bufs[bi]
        off = c * CHUNK
        pltpu.async_copy(rc.at[pl.ds(off, CHUNK)], cb, sem)
        pltpu.async_copy(vals.at[pl.ds(off, CHUNK)], vb, sem)

    def _wait(bi):
        cb, vb, sem = bufs[bi]
        pltpu.make_async_copy(rc.at[pl.ds(0, CHUNK)], cb, sem).wait()
        pltpu.make_async_copy(vals.at[pl.ds(0, CHUNK)], vb, sem).wait()

    tcopies = [
        pltpu.async_copy(hp.at[pl.ds(pbase + p * N_NODES, N_NODES)], tables[p], tsem)
        for p in range(PPT)
    ]
    _start(0, 0)

    zeros = jnp.zeros((L,), jnp.float32)

    @plsc.parallel_loop(0, N_NODES // L, 1, unroll=8)
    def _zero(i):
        for d in range(DPT):
            accs[d][pl.ds(i * L, L)] = zeros

    for c in tcopies:
        c.wait()

    lowmask = jnp.full((L,), 0x3FFF, jnp.int32)
    himask = jnp.full((L,), -65536, jnp.int32)

    def _process(bi):
        cb, vb, _ = bufs[bi]

        @plsc.parallel_loop(0, NBLK, 1, unroll=4)
        def _blk(b):
            s = b * L
            cr = cb[pl.ds(s, L)]
            vv = vb[pl.ds(s, L)]
            ci = cr & lowmask
            ri = lax.shift_right_logical(cr, 14)
            for p in range(PPT):
                g = plsc.load_gather(tables[p], [ci])
                glo = plsc.bitcast(lax.shift_left(g, 16), jnp.float32)
                ghi = plsc.bitcast(g & himask, jnp.float32)
                plsc.addupdate_scatter(accs[2 * p], [ri], glo * vv)
                plsc.addupdate_scatter(accs[2 * p + 1], [ri], ghi * vv)

    def _outer(cc, carry):
        c0 = cc * 2
        _start(c0 + 1, 1)
        _wait(0)
        _process(0)

        @pl.when(c0 + 2 < NCHUNK)
        def _():
            _start(c0 + 2, 0)

        _wait(1)
        _process(1)
        return carry

    lax.fori_loop(0, NCHUNK // 2, _outer, 0)

    for d in range(DPT):
        pltpu.sync_copy(accs[d], out.at[pl.ds(acc_dims[d] * N_NODES, N_NODES)])


def _pack_bf16_pair(ulo, uhi):
    plo = lax.shift_right_logical(ulo + jnp.uint32(0x8000), jnp.uint32(16))
    phi = (uhi + jnp.uint32(0x8000)) & jnp.uint32(0xFFFF0000)
    return lax.bitcast_convert_type(phi | plo, jnp.int32)


def _tc_linear_relu_packT(aggT, Wlo, Whi, blo, bhi):
    def body(agg_ref, wlo_ref, whi_ref, blo_ref, bhi_ref, out_ref):
        ylo = jnp.maximum(
            lax.dot_general(wlo_ref[...], agg_ref[...], (((1,), (0,)), ((), ())),
                            preferred_element_type=jnp.float32) + blo_ref[...], 0.0)
        yhi = jnp.maximum(
            lax.dot_general(whi_ref[...], agg_ref[...], (((1,), (0,)), ((), ())),
                            preferred_element_type=jnp.float32) + bhi_ref[...], 0.0)
        out_ref[...] = _pack_bf16_pair(
            lax.bitcast_convert_type(ylo, jnp.uint32),
            lax.bitcast_convert_type(yhi, jnp.uint32))

    return pl.pallas_call(
        body,
        out_shape=jax.ShapeDtypeStruct((DIM // 2, N_NODES), jnp.int32),
    )(aggT, Wlo, Whi, blo, bhi)


def _tc_linear_relu_final(aggT, W, b_row):
    def body(agg_ref, w_ref, b_ref, out_ref):
        acc = lax.dot_general(
            agg_ref[...], w_ref[...], (((0,), (1,)), ((), ())),
            preferred_element_type=jnp.float32)
        out_ref[...] = jnp.maximum(acc + b_ref[...], 0.0)

    return pl.pallas_call(
        body,
        out_shape=jax.ShapeDtypeStruct((N_NODES, DIM), jnp.float32),
    )(aggT, W, b_row)


def kernel(edge_index, adj_vals, emb, W1, b1, W2, b2):
    row = edge_index[0].astype(jnp.int32)
    col = edge_index[1].astype(jnp.int32)
    rc = (row << 14) | col

    u = lax.bitcast_convert_type(emb, jnp.uint32)
    hp1 = _pack_bf16_pair(u[:, :DIM // 2], u[:, DIM // 2:]).T

    agg1 = _sc_aggregate(rc, adj_vals, hp1.reshape(-1)).reshape(DIM, N_NODES)
    hp2 = _tc_linear_relu_packT(agg1, W1[:DIM // 2], W1[DIM // 2:],
                                b1[:DIM // 2].reshape(DIM // 2, 1),
                                b1[DIM // 2:].reshape(DIM // 2, 1))
    agg2 = _sc_aggregate(rc, adj_vals, hp2.reshape(-1)).reshape(DIM, N_NODES)
    out = _tc_linear_relu_final(agg2, W2, b2.reshape(1, DIM))
    return out
